# baseline (speedup 1.0000x reference)
import jax
import jax.numpy as jnp
import numpy as np
from jax import lax
from jax.experimental import pallas as pl
from jax.experimental.pallas import tpu as pltpu
from jax.experimental.pallas import tpu_sc as plsc

_N_CP = 20
_B = 16384
_NC = 2
_NS = 16
_L = 16
_NW = _NC * _NS
_BPW = _B // _NW
_CHUNKS = _BPW // _L

_S = np.linspace(0.0, 800.0, _N_CP + 1)[1:].astype(np.float32)

_mesh = plsc.VectorSubcoreMesh(
    core_axis_name="c", subcore_axis_name="s", num_cores=_NC, num_subcores=_NS
)

_SCRATCH = [
    pltpu.VMEM((_BPW,), jnp.int32),
    pltpu.VMEM((_BPW,), jnp.float32),
    pltpu.VMEM((_BPW * _N_CP,), jnp.int32),
    pltpu.VMEM((_BPW * _N_CP,), jnp.float32),
    pltpu.VMEM((_BPW,), jnp.float32),
    pltpu.VMEM((_BPW,), jnp.float32),
    pltpu.VMEM((_BPW,), jnp.float32),
    pltpu.SemaphoreType.DMA,
    pltpu.SemaphoreType.DMA,
    pltpu.SemaphoreType.DMA,
]


def _trend_body(t_hbm, idx_hbm, m_hbm, k_hbm, delta_hbm, out_hbm,
                idx_v, t_v, ix2_v, d_v, m_v, k_v, o_v, sem_d, sem_m, sem_k):
    wid = lax.axis_index("s") * _NC + lax.axis_index("c")
    base = wid * _BPW

    pltpu.sync_copy(idx_hbm.at[pl.ds(base, _BPW)], idx_v)
    cm = pltpu.async_copy(m_hbm.at[idx_v], m_v, sem_m)
    ck = pltpu.async_copy(k_hbm.at[idx_v], k_v, sem_k)

    def expand(c, carry):
        o = c * _L
        iv = idx_v[pl.ds(o, _L)] * _N_CP
        for j in range(_N_CP):
            ix2_v[pl.ds(j * _BPW + o, _L)] = iv + j
        return carry

    lax.fori_loop(0, _CHUNKS, expand, 0)
    cd = pltpu.async_copy(delta_hbm.at[ix2_v], d_v, sem_d)
    pltpu.sync_copy(t_hbm.at[pl.ds(base, _BPW)], t_v)
    cm.wait()
    ck.wait()
    cd.wait()

    def body(c, carry):
        o = c * _L
        tv = t_v[pl.ds(o, _L)]
        acc = m_v[pl.ds(o, _L)] + k_v[pl.ds(o, _L)] * tv
        for j in range(_N_CP):
            w = jnp.maximum(tv - _S[j], 0.0)
            acc = acc + w * d_v[pl.ds(j * _BPW + o, _L)]
        o_v[pl.ds(o, _L)] = acc
        return carry

    lax.fori_loop(0, _CHUNKS, body, 0)
    pltpu.sync_copy(o_v, out_hbm.at[pl.ds(base, _BPW)])


_trend_sc = pl.kernel(
    _trend_body,
    out_type=jax.ShapeDtypeStruct((_B,), jnp.float32),
    mesh=_mesh,
    compiler_params=pltpu.CompilerParams(
        needs_layout_passes=False, use_tc_tiling_on_sc=False
    ),
    scratch_types=_SCRATCH,
)


def kernel(t, idx, m_table, k_table, delta_table):
    tf = t.reshape(-1).astype(jnp.float32)
    idxf = idx.reshape(-1).astype(jnp.int32)
    mf = m_table.reshape(-1)
    kf = k_table.reshape(-1)
    df = delta_table.reshape(-1)
    out = _trend_sc(tf, idxf, mf, kf, df)
    return out.reshape(-1, 1)


# speedup vs baseline: 1.0124x; 1.0124x over previous
import jax
import jax.numpy as jnp
import numpy as np
from jax import lax
from jax.experimental import pallas as pl
from jax.experimental.pallas import tpu as pltpu
from jax.experimental.pallas import tpu_sc as plsc

_N_CP = 20
_B = 16384
_NC = 2
_NS = 16
_L = 16
_NW = _NC * _NS
_BPW = _B // _NW
_CHUNKS = _BPW // _L
_R16 = (1000000 * _N_CP) // 16

_S = np.linspace(0.0, 800.0, _N_CP + 1)[1:].astype(np.float32)

_mesh = plsc.VectorSubcoreMesh(
    core_axis_name="c", subcore_axis_name="s", num_cores=_NC, num_subcores=_NS
)

_SCRATCH = [
    pltpu.VMEM((_BPW,), jnp.int32),
    pltpu.VMEM((_BPW,), jnp.float32),
    pltpu.VMEM((2 * _BPW,), jnp.int32),
    pltpu.VMEM((_BPW,), jnp.int32),
    pltpu.VMEM((2 * _BPW, 16), jnp.float32),
    pltpu.VMEM((_BPW,), jnp.float32),
    pltpu.VMEM((_BPW,), jnp.float32),
    pltpu.VMEM((_BPW,), jnp.float32),
    pltpu.SemaphoreType.DMA,
    pltpu.SemaphoreType.DMA,
    pltpu.SemaphoreType.DMA,
]


def _trend_body(t_hbm, idx_hbm, m_hbm, k_hbm, delta16_hbm, out_hbm,
                idx_v, t_v, ir_v, p_v, pair_v, m_v, k_v, o_v,
                sem_d, sem_m, sem_k):
    wid = lax.axis_index("s") * _NC + lax.axis_index("c")
    base = wid * _BPW

    pltpu.sync_copy(idx_hbm.at[pl.ds(base, _BPW)], idx_v)
    cm = pltpu.async_copy(m_hbm.at[idx_v], m_v, sem_m)
    ck = pltpu.async_copy(k_hbm.at[idx_v], k_v, sem_k)

    def expand(c, carry):
        o = c * _L
        iv = idx_v[pl.ds(o, _L)]
        r0 = (iv * 5) >> 2
        ir_v[pl.ds(o, _L)] = r0
        ir_v[pl.ds(_BPW + o, _L)] = r0 + 1
        p_v[pl.ds(o, _L)] = (iv & 3) << 2
        return carry

    lax.fori_loop(0, _CHUNKS, expand, 0)
    cd = pltpu.async_copy(delta16_hbm.at[ir_v], pair_v, sem_d)
    pltpu.sync_copy(t_hbm.at[pl.ds(base, _BPW)], t_v)
    cm.wait()
    ck.wait()
    cd.wait()

    def body(c, carry):
        o = c * _L
        rows = o + lax.iota(jnp.int32, _L)
        tv = t_v[pl.ds(o, _L)]
        pv = p_v[pl.ds(o, _L)]
        acc = m_v[pl.ds(o, _L)] + k_v[pl.ds(o, _L)] * tv
        for j in range(_N_CP):
            w = jnp.maximum(tv - _S[j], 0.0)
            pj = pv + j
            r = rows + ((pj >> 4) << 9)
            d = plsc.load_gather(pair_v, [r, pj & 15])
            acc = acc + w * d
        o_v[pl.ds(o, _L)] = acc
        return carry

    lax.fori_loop(0, _CHUNKS, body, 0)
    pltpu.sync_copy(o_v, out_hbm.at[pl.ds(base, _BPW)])


_trend_sc = pl.kernel(
    _trend_body,
    out_type=jax.ShapeDtypeStruct((_B,), jnp.float32),
    mesh=_mesh,
    compiler_params=pltpu.CompilerParams(
        needs_layout_passes=False, use_tc_tiling_on_sc=False
    ),
    scratch_types=_SCRATCH,
)


def kernel(t, idx, m_table, k_table, delta_table):
    tf = t.reshape(-1).astype(jnp.float32)
    idxf = idx.reshape(-1).astype(jnp.int32)
    mf = m_table.reshape(-1)
    kf = k_table.reshape(-1)
    d16 = delta_table.reshape(_R16, 16)
    out = _trend_sc(tf, idxf, mf, kf, d16)
    return out.reshape(-1, 1)
